# Initial kernel scaffold; baseline (speedup 1.0000x reference)
import jax
import jax.numpy as jnp
from jax import lax
from jax.experimental import pallas as pl
from jax.experimental.pallas import tpu as pltpu
from jax.experimental.pallas import tpu_sc as plsc

N = 10000
E = 320000
D = 128
NC = 2
NS = 16
NW = NC * NS
EPW = E // NW
C = 80
NCHUNK = EPW // C
RPT = N // NS
HW = 16

_MESH = plsc.VectorSubcoreMesh(core_axis_name="c", subcore_axis_name="s")


def _deg_body(dst3, ones, zeros, deg_out, hist, idx_v, ones_v):
    c = lax.axis_index("c")
    s = lax.axis_index("s")
    wid = s * NC + c
    pltpu.sync_copy(dst3.at[wid], idx_v)
    pltpu.sync_copy(ones, ones_v)
    pltpu.sync_copy(zeros.at[pl.ds(s * RPT, RPT)], hist.at[pl.ds(s * RPT, RPT)])
    plsc.subcore_barrier()

    def body(j, carry):
        pltpu.sync_copy(ones_v, hist.at[idx_v.at[j]], add=True)
        return carry

    lax.fori_loop(0, NCHUNK, body, 0)
    plsc.subcore_barrier()
    pltpu.sync_copy(hist.at[pl.ds(s * RPT, RPT)],
                    deg_out.at[c].at[pl.ds(s * RPT, RPT)])


_deg_call = pl.kernel(
    _deg_body,
    out_type=jax.ShapeDtypeStruct((NC, N, HW), jnp.float32),
    mesh=_MESH,
    scratch_types=[
        pltpu.VMEM_SHARED((N, HW), jnp.float32),
        pltpu.VMEM((NCHUNK, C), jnp.int32),
        pltpu.VMEM((C, HW), jnp.float32),
    ],
)


def _agg_body(xp, src3, dst3, zeros, agg_out, acc, sidx, didx, rows, sem):
    c = lax.axis_index("c")
    s = lax.axis_index("s")
    wid = s * NC + c
    pltpu.sync_copy(src3.at[wid], sidx)
    pltpu.sync_copy(dst3.at[wid], didx)
    pltpu.sync_copy(zeros.at[pl.ds(s * RPT, RPT)], acc.at[pl.ds(s * RPT, RPT)])
    plsc.subcore_barrier()

    def body(j, carry):
        pltpu.async_copy(xp.at[sidx.at[j]], rows, sem).wait()
        pltpu.sync_copy(rows, acc.at[didx.at[j]], add=True)
        return carry

    lax.fori_loop(0, NCHUNK, body, 0)
    plsc.subcore_barrier()
    pltpu.sync_copy(acc.at[pl.ds(s * RPT, RPT)],
                    agg_out.at[c].at[pl.ds(s * RPT, RPT)])


_agg_call = pl.kernel(
    _agg_body,
    out_type=jax.ShapeDtypeStruct((NC, N, D), jnp.float32),
    mesh=_MESH,
    scratch_types=[
        pltpu.VMEM_SHARED((N, D), jnp.float32),
        pltpu.VMEM((NCHUNK, C), jnp.int32),
        pltpu.VMEM((NCHUNK, C), jnp.int32),
        pltpu.VMEM((C, D), jnp.float32),
        pltpu.SemaphoreType.DMA,
    ],
)


def _prep_body(deg_ref, x_ref, xp_ref, norm_ref):
    d = deg_ref[0, :, 0:1] + deg_ref[1, :, 0:1]
    norm = lax.rsqrt(jnp.maximum(d, 1.0))
    norm_ref[...] = norm
    xp_ref[...] = x_ref[...] * norm


_prep_call = pl.pallas_call(
    _prep_body,
    out_shape=(
        jax.ShapeDtypeStruct((N, D), jnp.float32),
        jax.ShapeDtypeStruct((N, 1), jnp.float32),
    ),
)


def _fin_body(agg_ref, norm_ref, w_ref, b_ref, o_ref):
    a = (agg_ref[0] + agg_ref[1]) * norm_ref[...]
    acc = jnp.dot(a, w_ref[...], preferred_element_type=jnp.float32)
    o_ref[...] = jnp.maximum(acc + b_ref[...], 0.0)


_fin_call = pl.pallas_call(
    _fin_body,
    out_shape=jax.ShapeDtypeStruct((N, D), jnp.float32),
)


def kernel(t, x, edge_index, W, b):
    src3 = edge_index[0].reshape(NW, NCHUNK, C)
    dst3 = edge_index[1].reshape(NW, NCHUNK, C)
    ones = jnp.ones((C, HW), jnp.float32)
    zeros_h = jnp.zeros((N, HW), jnp.float32)
    zeros_a = jnp.zeros((N, D), jnp.float32)
    deg = _deg_call(dst3, ones, zeros_h)
    xp, norm = _prep_call(deg, x)
    agg2 = _agg_call(xp, src3, dst3, zeros_a)
    return _fin_call(agg2, norm, W, b.reshape(1, D))


# speedup vs baseline: 15.7544x; 15.7544x over previous
import jax
import jax.numpy as jnp
from jax import lax
from jax.experimental import pallas as pl
from jax.experimental.pallas import tpu as pltpu
from jax.experimental.pallas import tpu_sc as plsc

N = 10000
E = 320000
D = 128
NC = 2
NS = 16
NW = NC * NS
EPW = E // NW
C = 80
NCHUNK = EPW // C
NP = 10240
RPT = NP // NS
HW = 16

_MESH = plsc.VectorSubcoreMesh(core_axis_name="c", subcore_axis_name="s")


def _deg_body(dst3, ones, zeros, deg_out, hist, idx_v, ones_v):
    c = lax.axis_index("c")
    s = lax.axis_index("s")
    wid = s * NC + c
    pltpu.sync_copy(dst3.at[wid], idx_v)
    pltpu.sync_copy(ones, ones_v)
    pltpu.sync_copy(zeros.at[pl.ds(s * RPT, RPT)], hist.at[pl.ds(s * RPT, RPT)])
    plsc.subcore_barrier()

    def body(j, carry):
        pltpu.sync_copy(ones_v, hist.at[idx_v.at[j]], add=True)
        return carry

    lax.fori_loop(0, NCHUNK, body, 0)
    plsc.subcore_barrier()
    pltpu.sync_copy(hist.at[pl.ds(s * RPT, RPT)],
                    deg_out.at[c].at[pl.ds(s * RPT, RPT)])


_deg_call = pl.kernel(
    _deg_body,
    out_type=jax.ShapeDtypeStruct((NC, NP), jnp.float32),
    mesh=_MESH,
    scratch_types=[
        pltpu.VMEM_SHARED((NP,), jnp.float32),
        pltpu.VMEM((NCHUNK, C), jnp.int32),
        pltpu.VMEM((C,), jnp.float32),
    ],
)


def _agg_body(xp, src3, dst3, zeros, agg_out, acc, sidx, didx, rows, sem):
    c = lax.axis_index("c")
    s = lax.axis_index("s")
    wid = s * NC + c
    pltpu.sync_copy(src3.at[wid], sidx)
    pltpu.sync_copy(dst3.at[wid], didx)
    pltpu.sync_copy(zeros.at[pl.ds(s * RPT, RPT)], acc.at[pl.ds(s * RPT, RPT)])
    plsc.subcore_barrier()

    def body(j, carry):
        pltpu.async_copy(xp.at[sidx.at[j]], rows, sem).wait()
        pltpu.sync_copy(rows, acc.at[didx.at[j]], add=True)
        return carry

    lax.fori_loop(0, NCHUNK, body, 0)
    plsc.subcore_barrier()
    pltpu.sync_copy(acc.at[pl.ds(s * RPT, RPT)],
                    agg_out.at[c].at[pl.ds(s * RPT, RPT)])


_agg_call = pl.kernel(
    _agg_body,
    out_type=jax.ShapeDtypeStruct((NC, NP, D), jnp.float32),
    mesh=_MESH,
    scratch_types=[
        pltpu.VMEM_SHARED((NP, D), jnp.float32),
        pltpu.VMEM((NCHUNK, C), jnp.int32),
        pltpu.VMEM((NCHUNK, C), jnp.int32),
        pltpu.VMEM((C, D), jnp.float32),
        pltpu.SemaphoreType.DMA,
    ],
)


def _prep_body(deg_ref, x_ref, xp_ref, norm_ref):
    d = (deg_ref[0, :N] + deg_ref[1, :N])[:, None]
    norm = lax.rsqrt(jnp.maximum(d, 1.0))
    norm_ref[...] = norm
    xp_ref[...] = x_ref[...] * norm


_prep_call = pl.pallas_call(
    _prep_body,
    out_shape=(
        jax.ShapeDtypeStruct((N, D), jnp.float32),
        jax.ShapeDtypeStruct((N, 1), jnp.float32),
    ),
)


def _fin_body(agg_ref, norm_ref, w_ref, b_ref, o_ref):
    a = (agg_ref[0, :N] + agg_ref[1, :N]) * norm_ref[...]
    acc = jnp.dot(a, w_ref[...], preferred_element_type=jnp.float32)
    o_ref[...] = jnp.maximum(acc + b_ref[...], 0.0)


_fin_call = pl.pallas_call(
    _fin_body,
    out_shape=jax.ShapeDtypeStruct((N, D), jnp.float32),
)


def kernel(t, x, edge_index, W, b):
    src3 = edge_index[0].reshape(NW, NCHUNK, C)
    dst3 = edge_index[1].reshape(NW, NCHUNK, C)
    ones = jnp.ones((C,), jnp.float32)
    zeros_h = jnp.zeros((NP,), jnp.float32)
    zeros_a = jnp.zeros((NP, D), jnp.float32)
    deg = _deg_call(dst3, ones, zeros_h)
    xp, norm = _prep_call(deg, x)
    agg2 = _agg_call(xp, src3, dst3, zeros_a)
    return _fin_call(agg2, norm, W, b.reshape(1, D))


# speedup vs baseline: 22.4286x; 1.4236x over previous
import jax
import jax.numpy as jnp
from jax import lax
from jax.experimental import pallas as pl
from jax.experimental.pallas import tpu as pltpu
from jax.experimental.pallas import tpu_sc as plsc

N = 10000
E = 320000
D = 128
NC = 2
NS = 16
NW = NC * NS
EPW = E // NW
C = 80
NCHUNK = EPW // C
NP = 10240
RPT = NP // NS
HW = 16

_MESH = plsc.VectorSubcoreMesh(core_axis_name="c", subcore_axis_name="s")


def _deg_body(dst3, ones, zeros, deg_out, hist, idx_v, ones_v):
    c = lax.axis_index("c")
    s = lax.axis_index("s")
    wid = s * NC + c
    pltpu.sync_copy(dst3.at[wid], idx_v)
    pltpu.sync_copy(ones, ones_v)
    pltpu.sync_copy(zeros.at[pl.ds(s * RPT, RPT)], hist.at[pl.ds(s * RPT, RPT)])
    plsc.subcore_barrier()

    def body(j, carry):
        pltpu.sync_copy(ones_v, hist.at[idx_v.at[j]], add=True)
        return carry

    lax.fori_loop(0, NCHUNK, body, 0)
    plsc.subcore_barrier()
    pltpu.sync_copy(hist.at[pl.ds(s * RPT, RPT)],
                    deg_out.at[c].at[pl.ds(s * RPT, RPT)])


_deg_call = pl.kernel(
    _deg_body,
    out_type=jax.ShapeDtypeStruct((NC, NP), jnp.float32),
    mesh=_MESH,
    scratch_types=[
        pltpu.VMEM_SHARED((NP,), jnp.float32),
        pltpu.VMEM((NCHUNK, C), jnp.int32),
        pltpu.VMEM((C,), jnp.float32),
    ],
)


CB = 128
EPWP = 10240
NCH = EPWP // CB
NPAD = EPWP - EPW


def _agg_body(idxp, xp, zeros, agg_out, acc, ibuf, rows,
              semi0, semi1, semg0, semg1):
    c = lax.axis_index("c")
    s = lax.axis_index("s")
    wid = s * NC + c
    semi = (semi0, semi1)
    semg = (semg0, semg1)
    pltpu.sync_copy(zeros.at[pl.ds(s * RPT, RPT)], acc.at[pl.ds(s * RPT, RPT)])
    plsc.subcore_barrier()

    pltpu.async_copy(idxp.at[wid].at[0], ibuf.at[0], semi[0])
    pltpu.async_copy(idxp.at[wid].at[1], ibuf.at[1], semi[1])
    pltpu.make_async_copy(idxp.at[wid].at[0], ibuf.at[0], semi[0]).wait()
    pltpu.async_copy(xp.at[ibuf.at[0].at[0]], rows.at[0], semg[0])

    def body(g, carry):
        for b in range(2):
            ch = g * 2 + b
            b1 = 1 - b

            @pl.when(ch + 1 < NCH)
            def _():
                pltpu.make_async_copy(idxp.at[wid].at[ch + 1], ibuf.at[b1],
                                      semi[b1]).wait()
                pltpu.async_copy(xp.at[ibuf.at[b1].at[0]], rows.at[b1],
                                 semg[b1])

            pltpu.make_async_copy(xp.at[ibuf.at[b].at[0]], rows.at[b],
                                  semg[b]).wait()
            pltpu.sync_copy(rows.at[b], acc.at[ibuf.at[b].at[1]], add=True)

            @pl.when(ch + 2 < NCH)
            def _():
                pltpu.async_copy(idxp.at[wid].at[ch + 2], ibuf.at[b],
                                 semi[b])
        return carry

    lax.fori_loop(0, NCH // 2, body, 0)
    plsc.subcore_barrier()
    pltpu.sync_copy(acc.at[pl.ds(s * RPT, RPT)],
                    agg_out.at[c].at[pl.ds(s * RPT, RPT)])


_agg_call = pl.kernel(
    _agg_body,
    out_type=jax.ShapeDtypeStruct((NC, NP, D), jnp.float32),
    mesh=_MESH,
    scratch_types=[
        pltpu.VMEM_SHARED((NP, D), jnp.float32),
        pltpu.VMEM((2, 2, CB), jnp.int32),
        pltpu.VMEM((2, CB, D), jnp.float32),
        pltpu.SemaphoreType.DMA,
        pltpu.SemaphoreType.DMA,
        pltpu.SemaphoreType.DMA,
        pltpu.SemaphoreType.DMA,
    ],
)


def _prep_body(deg_ref, x_ref, xp_ref, norm_ref):
    d = (deg_ref[0, :N] + deg_ref[1, :N])[:, None]
    norm = lax.rsqrt(jnp.maximum(d, 1.0))
    norm_ref[...] = norm
    xp_ref[...] = x_ref[...] * norm


_prep_call = pl.pallas_call(
    _prep_body,
    out_shape=(
        jax.ShapeDtypeStruct((N, D), jnp.float32),
        jax.ShapeDtypeStruct((N, 1), jnp.float32),
    ),
)


def _fin_body(agg_ref, norm_ref, w_ref, b_ref, o_ref):
    a = (agg_ref[0, :N] + agg_ref[1, :N]) * norm_ref[...]
    acc = jnp.dot(a, w_ref[...], preferred_element_type=jnp.float32)
    o_ref[...] = jnp.maximum(acc + b_ref[...], 0.0)


_fin_call = pl.pallas_call(
    _fin_body,
    out_shape=jax.ShapeDtypeStruct((N, D), jnp.float32),
)


def kernel(t, x, edge_index, W, b):
    dst3 = edge_index[1].reshape(NW, NCHUNK, C)
    pad_src = jnp.broadcast_to((jnp.arange(NPAD, dtype=jnp.int32) * 37) % N,
                               (NW, NPAD))
    pad_dst = jnp.broadcast_to(N + jnp.arange(NPAD, dtype=jnp.int32),
                               (NW, NPAD))
    src_p = jnp.concatenate([edge_index[0].reshape(NW, EPW), pad_src], axis=1)
    dst_p = jnp.concatenate([edge_index[1].reshape(NW, EPW), pad_dst], axis=1)
    idxp = jnp.stack([src_p.reshape(NW, NCH, CB),
                      dst_p.reshape(NW, NCH, CB)], axis=2)
    ones = jnp.ones((C,), jnp.float32)
    zeros_h = jnp.zeros((NP,), jnp.float32)
    zeros_a = jnp.zeros((NP, D), jnp.float32)
    deg = _deg_call(dst3, ones, zeros_h)
    xp, norm = _prep_call(deg, x)
    agg2 = _agg_call(idxp, xp, zeros_a)
    return _fin_call(agg2, norm, W, b.reshape(1, D))


# speedup vs baseline: 22.9458x; 1.0231x over previous
import jax
import jax.numpy as jnp
from jax import lax
from jax.experimental import pallas as pl
from jax.experimental.pallas import tpu as pltpu
from jax.experimental.pallas import tpu_sc as plsc

N = 10000
E = 320000
D = 128
NC = 2
NS = 16
NW = NC * NS
CB = 128
NCHT = E // CB
CHW = NCHT // NW
CXT = NCHT - CHW * NW
NP = 10240
RPT = NP // NS

_MESH = plsc.VectorSubcoreMesh(core_axis_name="c", subcore_axis_name="s")


def _wid_base_cnt():
    c = lax.axis_index("c")
    s = lax.axis_index("s")
    wid = s * NC + c
    base = CHW * wid + jnp.minimum(wid, CXT)
    cnt = CHW + jnp.where(wid < CXT, 1, 0)
    return c, s, wid, base, cnt


def _deg_body(ei3, ones, zeros, deg_out, hist, dbuf, ones_v, semi0, semi1):
    c, s, wid, base, cnt = _wid_base_cnt()
    semi = (semi0, semi1)
    pltpu.sync_copy(ones, ones_v)
    pltpu.sync_copy(zeros, hist.at[pl.ds(s * RPT, RPT)])
    plsc.subcore_barrier()

    pltpu.async_copy(ei3.at[1].at[base], dbuf.at[0], semi[0])
    pltpu.async_copy(ei3.at[1].at[base + 1], dbuf.at[1], semi[1])

    def body(g, carry):
        for b in range(2):
            k = g * 2 + b
            pltpu.make_async_copy(ei3.at[1].at[base + k], dbuf.at[b],
                                  semi[b]).wait()
            pltpu.sync_copy(ones_v, hist.at[dbuf.at[b]], add=True)

            @pl.when(k + 2 < cnt)
            def _():
                pltpu.async_copy(ei3.at[1].at[base + k + 2], dbuf.at[b],
                                 semi[b])
        return carry

    lax.fori_loop(0, CHW // 2, body, 0)

    @pl.when(wid < CXT)
    def _():
        b = CHW % 2
        pltpu.make_async_copy(ei3.at[1].at[base + CHW], dbuf.at[b],
                              semi[b]).wait()
        pltpu.sync_copy(ones_v, hist.at[dbuf.at[b]], add=True)

    plsc.subcore_barrier()
    pltpu.sync_copy(hist.at[pl.ds(s * RPT, RPT)],
                    deg_out.at[c].at[pl.ds(s * RPT, RPT)])


_deg_call = pl.kernel(
    _deg_body,
    out_type=jax.ShapeDtypeStruct((NC, NP), jnp.float32),
    mesh=_MESH,
    scratch_types=[
        pltpu.VMEM_SHARED((NP,), jnp.float32),
        pltpu.VMEM((2, CB), jnp.int32),
        pltpu.VMEM((CB,), jnp.float32),
        pltpu.SemaphoreType.DMA,
        pltpu.SemaphoreType.DMA,
    ],
)


def _agg_body(ei3, xp, zeros, agg_out, acc, sbuf, dbuf, rows,
              semi0, semi1, semg0, semg1):
    c, s, wid, base, cnt = _wid_base_cnt()
    semi = (semi0, semi1)
    semg = (semg0, semg1)
    pltpu.sync_copy(zeros, acc.at[pl.ds(s * RPT, RPT)])
    plsc.subcore_barrier()

    def load_idx(k, b):
        pltpu.async_copy(ei3.at[0].at[base + k], sbuf.at[b], semi[b])
        pltpu.async_copy(ei3.at[1].at[base + k], dbuf.at[b], semi[b])

    def wait_idx(k, b):
        pltpu.make_async_copy(ei3.at[0].at[base + k], sbuf.at[b],
                              semi[b]).wait()
        pltpu.make_async_copy(ei3.at[1].at[base + k], dbuf.at[b],
                              semi[b]).wait()

    def fire_gather(b):
        pltpu.async_copy(xp.at[sbuf.at[b]], rows.at[b], semg[b])

    def wait_gather(b):
        pltpu.make_async_copy(xp.at[sbuf.at[b]], rows.at[b], semg[b]).wait()

    load_idx(0, 0)
    load_idx(1, 1)
    wait_idx(0, 0)
    fire_gather(0)

    def body(g, carry):
        for b in range(2):
            k = g * 2 + b
            b1 = 1 - b

            @pl.when(k + 1 < cnt)
            def _():
                wait_idx(k + 1, b1)
                fire_gather(b1)

            wait_gather(b)
            pltpu.sync_copy(rows.at[b], acc.at[dbuf.at[b]], add=True)

            @pl.when(k + 2 < cnt)
            def _():
                load_idx(k + 2, b)
        return carry

    lax.fori_loop(0, CHW // 2, body, 0)

    @pl.when(wid < CXT)
    def _():
        b = CHW % 2
        wait_gather(b)
        pltpu.sync_copy(rows.at[b], acc.at[dbuf.at[b]], add=True)

    plsc.subcore_barrier()
    pltpu.sync_copy(acc.at[pl.ds(s * RPT, RPT)],
                    agg_out.at[c].at[pl.ds(s * RPT, RPT)])


_agg_call = pl.kernel(
    _agg_body,
    out_type=jax.ShapeDtypeStruct((NC, NP, D), jnp.float32),
    mesh=_MESH,
    scratch_types=[
        pltpu.VMEM_SHARED((NP, D), jnp.float32),
        pltpu.VMEM((2, CB), jnp.int32),
        pltpu.VMEM((2, CB), jnp.int32),
        pltpu.VMEM((2, CB, D), jnp.float32),
        pltpu.SemaphoreType.DMA,
        pltpu.SemaphoreType.DMA,
        pltpu.SemaphoreType.DMA,
        pltpu.SemaphoreType.DMA,
    ],
)


def _prep_body(deg_ref, x_ref, xp_ref, norm_ref):
    d = (deg_ref[0, :N] + deg_ref[1, :N])[:, None]
    norm = lax.rsqrt(jnp.maximum(d, 1.0))
    norm_ref[...] = norm
    xp_ref[...] = x_ref[...] * norm


_prep_call = pl.pallas_call(
    _prep_body,
    out_shape=(
        jax.ShapeDtypeStruct((N, D), jnp.float32),
        jax.ShapeDtypeStruct((N, 1), jnp.float32),
    ),
)


def _fin_body(agg_ref, norm_ref, w_ref, b_ref, o_ref):
    a = (agg_ref[0, :N] + agg_ref[1, :N]) * norm_ref[...]
    acc = jnp.dot(a, w_ref[...], preferred_element_type=jnp.float32)
    o_ref[...] = jnp.maximum(acc + b_ref[...], 0.0)


_fin_call = pl.pallas_call(
    _fin_body,
    out_shape=jax.ShapeDtypeStruct((N, D), jnp.float32),
)


def kernel(t, x, edge_index, W, b):
    ei3 = edge_index.reshape(2, NCHT, CB)
    ones = jnp.ones((CB,), jnp.float32)
    zeros_h = jnp.zeros((RPT,), jnp.float32)
    zeros_a = jnp.zeros((RPT, D), jnp.float32)
    deg = _deg_call(ei3, ones, zeros_h)
    xp, norm = _prep_call(deg, x)
    agg2 = _agg_call(ei3, xp, zeros_a)
    return _fin_call(agg2, norm, W, b.reshape(1, D))


# speedup vs baseline: 27.3455x; 1.1917x over previous
import jax
import jax.numpy as jnp
from jax import lax
from jax.experimental import pallas as pl
from jax.experimental.pallas import tpu as pltpu
from jax.experimental.pallas import tpu_sc as plsc

N = 10000
E = 320000
D = 128
NC = 2
NS = 16
NW = NC * NS
CB = 128
NCHT = E // CB
CHW = NCHT // NW
CXT = NCHT - CHW * NW
NP = 10240
RPT = NP // NS
NPA = 10112
RPA = NPA // NS
DEGQ = 6
AGQ = 3

_MESH = plsc.VectorSubcoreMesh(core_axis_name="c", subcore_axis_name="s")


def _wid_base_cnt():
    c = lax.axis_index("c")
    s = lax.axis_index("s")
    wid = s * NC + c
    base = CHW * wid + jnp.minimum(wid, CXT)
    cnt = CHW + jnp.where(wid < CXT, 1, 0)
    return c, s, wid, base, cnt


def _deg_body(ei3, ones, zeros, deg_out, hist, dbuf, ones_v, semi):
    c, s, wid, base, cnt = _wid_base_cnt()
    pltpu.sync_copy(ones, ones_v)
    pltpu.sync_copy(zeros, hist.at[pl.ds(s * RPT, RPT)])
    plsc.subcore_barrier()

    for b in range(DEGQ):
        pltpu.async_copy(ei3.at[1].at[base + b], dbuf.at[b], semi)

    def body(g, carry):
        for b in range(DEGQ):
            k = g * DEGQ + b
            pltpu.make_async_copy(ei3.at[1].at[base + k], dbuf.at[b],
                                  semi).wait()
            pltpu.sync_copy(ones_v, hist.at[dbuf.at[b]], add=True)

            @pl.when(k + DEGQ < cnt)
            def _():
                pltpu.async_copy(ei3.at[1].at[base + k + DEGQ], dbuf.at[b],
                                 semi)
        return carry

    lax.fori_loop(0, CHW // DEGQ, body, 0)

    @pl.when(wid < CXT)
    def _():
        b = CHW % DEGQ
        pltpu.make_async_copy(ei3.at[1].at[base + CHW], dbuf.at[b],
                              semi).wait()
        pltpu.sync_copy(ones_v, hist.at[dbuf.at[b]], add=True)

    plsc.subcore_barrier()
    pltpu.sync_copy(hist.at[pl.ds(s * RPT, RPT)],
                    deg_out.at[c].at[pl.ds(s * RPT, RPT)])


_deg_call = pl.kernel(
    _deg_body,
    out_type=jax.ShapeDtypeStruct((NC, NP), jnp.float32),
    mesh=_MESH,
    scratch_types=[
        pltpu.VMEM_SHARED((NP,), jnp.float32),
        pltpu.VMEM((DEGQ, CB), jnp.int32),
        pltpu.VMEM((CB,), jnp.float32),
        pltpu.SemaphoreType.DMA,
    ],
)


def _agg_body(ei3, xp, zeros, agg_out, acc, sbuf, dbuf, rows,
              semi0, semi1, semi2, semg0, semg1, semg2):
    c, s, wid, base, cnt = _wid_base_cnt()
    semi = (semi0, semi1, semi2)
    semg = (semg0, semg1, semg2)
    pltpu.sync_copy(zeros, acc.at[pl.ds(s * RPA, RPA)])
    plsc.subcore_barrier()

    def load_idx(k, b):
        pltpu.async_copy(ei3.at[0].at[base + k], sbuf.at[b], semi[b])
        pltpu.async_copy(ei3.at[1].at[base + k], dbuf.at[b], semi[b])

    def wait_idx(k, b):
        pltpu.make_async_copy(ei3.at[0].at[base + k], sbuf.at[b],
                              semi[b]).wait()
        pltpu.make_async_copy(ei3.at[1].at[base + k], dbuf.at[b],
                              semi[b]).wait()

    def fire_gather(b):
        pltpu.async_copy(xp.at[sbuf.at[b]], rows.at[b], semg[b])

    def wait_gather(b):
        pltpu.make_async_copy(xp.at[sbuf.at[b]], rows.at[b], semg[b]).wait()

    for b in range(AGQ):
        load_idx(b, b)
    wait_idx(0, 0)
    fire_gather(0)

    def body(g, carry):
        for b in range(AGQ):
            k = g * AGQ + b
            b1 = (b + 1) % AGQ

            @pl.when(k + 1 < cnt)
            def _():
                wait_idx(k + 1, b1)
                fire_gather(b1)

            wait_gather(b)
            pltpu.sync_copy(rows.at[b], acc.at[dbuf.at[b]], add=True)

            @pl.when(k + AGQ < cnt)
            def _():
                load_idx(k + AGQ, b)
        return carry

    lax.fori_loop(0, CHW // AGQ, body, 0)

    @pl.when(wid < CXT)
    def _():
        b = CHW % AGQ
        wait_gather(b)
        pltpu.sync_copy(rows.at[b], acc.at[dbuf.at[b]], add=True)

    plsc.subcore_barrier()
    pltpu.sync_copy(acc.at[pl.ds(s * RPA, RPA)],
                    agg_out.at[c].at[pl.ds(s * RPA, RPA)])


_agg_call = pl.kernel(
    _agg_body,
    out_type=jax.ShapeDtypeStruct((NC, NPA, D), jnp.float32),
    mesh=_MESH,
    scratch_types=[
        pltpu.VMEM_SHARED((NPA, D), jnp.float32),
        pltpu.VMEM((AGQ, CB), jnp.int32),
        pltpu.VMEM((AGQ, CB), jnp.int32),
        pltpu.VMEM((AGQ, CB, D), jnp.float32),
        pltpu.SemaphoreType.DMA,
        pltpu.SemaphoreType.DMA,
        pltpu.SemaphoreType.DMA,
        pltpu.SemaphoreType.DMA,
        pltpu.SemaphoreType.DMA,
        pltpu.SemaphoreType.DMA,
    ],
)


def _prep_body(deg_ref, x_ref, xp_ref, norm_ref):
    d = (deg_ref[0, :N] + deg_ref[1, :N])[:, None]
    norm = lax.rsqrt(jnp.maximum(d, 1.0))
    norm_ref[...] = norm
    xp_ref[...] = x_ref[...] * norm


_prep_call = pl.pallas_call(
    _prep_body,
    out_shape=(
        jax.ShapeDtypeStruct((N, D), jnp.float32),
        jax.ShapeDtypeStruct((N, 1), jnp.float32),
    ),
)


def _fin_body(agg_ref, norm_ref, w_ref, b_ref, o_ref):
    a = (agg_ref[0, :N] + agg_ref[1, :N]) * norm_ref[...]
    acc = jnp.dot(a, w_ref[...], preferred_element_type=jnp.float32)
    o_ref[...] = jnp.maximum(acc + b_ref[...], 0.0)


_fin_call = pl.pallas_call(
    _fin_body,
    out_shape=jax.ShapeDtypeStruct((N, D), jnp.float32),
)


def kernel(t, x, edge_index, W, b):
    ei3 = edge_index.reshape(2, NCHT, CB)
    ones = jnp.ones((CB,), jnp.float32)
    zeros_h = jnp.zeros((RPT,), jnp.float32)
    zeros_a = jnp.zeros((RPA, D), jnp.float32)
    deg = _deg_call(ei3, ones, zeros_h)
    xp, norm = _prep_call(deg, x)
    agg2 = _agg_call(ei3, xp, zeros_a)
    return _fin_call(agg2, norm, W, b.reshape(1, D))
